# baseline (speedup 1.0000x reference)
import functools

import jax
import jax.numpy as jnp
from jax import lax
from jax.experimental import pallas as pl
from jax.experimental.pallas import tpu as pltpu
from jax.experimental.pallas import tpu_sc as plsc

N = 10000
E = 160000
D = 384
DH = D // 2
OUT = 128
NC = 2
NS = 16
EB = 16
EPT = 10032
CHUNKS = EPT // EB
KK = CHUNKS // 3
E_PAD = NS * EPT
SLAB = 632
SLAB_LAST = N - (NS - 1) * SLAB
CNT_B = 40
CW = 8
CNT_E_PER_TILE = E // (NC * NS)
CNT_CHUNKS = CNT_E_PER_TILE // CNT_B
BN = 1000


def _make_agg():
    mesh = plsc.VectorSubcoreMesh(
        core_axis_name="c", subcore_axis_name="s", num_cores=NC, num_subcores=NS)
    out_type = jax.ShapeDtypeStruct((NC, N, DH), jnp.float32)
    bufset = [
        pltpu.VMEM((EB,), jnp.int32),
        pltpu.VMEM((EB,), jnp.int32),
        pltpu.VMEM((EB, 16), jnp.float32),
        pltpu.VMEM((EB, DH), jnp.float32),
        pltpu.VMEM((EB,), jnp.int32),
        pltpu.SemaphoreType.DMA,
        pltpu.SemaphoreType.DMA,
        pltpu.SemaphoreType.DMA,
    ]
    scratch = bufset * 3 + [pltpu.VMEM_SHARED((N, DH), jnp.float32)]

    @functools.partial(pl.kernel, out_type=out_type, mesh=mesh,
                       scratch_types=scratch,
                       compiler_params=pltpu.CompilerParams(
                           use_tc_tiling_on_sc=False,
                           needs_layout_passes=False))
    def agg(xs0, xs1, src_e, dst_e, ewr, z2d, s_out, *refs):
        sets = [refs[8 * i:8 * i + 8] for i in range(3)]
        acc = refs[24]
        c = lax.axis_index("c")
        s = lax.axis_index("s")
        r0 = s * SLAB

        for sz, cond in ((SLAB, s < NS - 1), (SLAB_LAST, s == NS - 1)):
            @pl.when(cond)
            def _(sz=sz):
                pltpu.sync_copy(z2d.at[pl.ds(r0, sz)], acc.at[pl.ds(r0, sz)])
        plsc.subcore_barrier()

        ebase = s * EPT

        def fire_idx(base, st):
            src_v, dst_v, ew_v = st[0], st[1], st[2]
            pltpu.async_copy(src_e.at[pl.ds(base, EB)], src_v, st[5])
            pltpu.async_copy(dst_e.at[pl.ds(base, EB)], dst_v, st[5])
            pltpu.async_copy(ewr.at[pl.ds(base, EB)], ew_v, st[5])

        def wait_idx(base, st):
            pltpu.make_async_copy(src_e.at[pl.ds(base, EB)], st[0], st[5]).wait()
            pltpu.make_async_copy(dst_e.at[pl.ds(base, EB)], st[1], st[5]).wait()
            pltpu.make_async_copy(ewr.at[pl.ds(base, EB)], st[2], st[5]).wait()

        def fire_gather(st):
            @pl.when(c == 0)
            def _():
                pltpu.async_copy(xs0.at[st[0]], st[3], st[6])

            @pl.when(c == 1)
            def _():
                pltpu.async_copy(xs1.at[st[0]], st[3], st[6])

        def wait_gather(st):
            pltpu.make_async_copy(xs0.at[st[0]], st[3], st[6]).wait()

        def fire_scatter(st):
            pltpu.async_copy(st[3], acc.at[st[4]], st[7], add=True)

        def wait_scatter(st):
            pltpu.make_async_copy(st[3], acc.at[st[4]], st[7]).wait()

        def shadow_scale(st):
            dst_v, ew_v, rows_v, shd_v = st[1], st[2], st[3], st[4]
            shd_v[pl.ds(0, 16)] = dst_v[pl.ds(0, 16)]
            for r in range(EB):
                m = ew_v[r, :]
                for q in range(DH // 16):
                    cs = pl.ds(q * 16, 16)
                    rows_v[r, cs] = rows_v[r, cs] * m

        fire_idx(ebase, sets[0])
        fire_idx(ebase + EB, sets[1])
        fire_idx(ebase + 2 * EB, sets[2])
        wait_idx(ebase, sets[0])
        fire_gather(sets[0])

        LAST = CHUNKS - 1

        def step(kk, carry):
            for p in range(3):
                j, j1 = p, (p + 1) % 3
                cc = 3 * kk + p
                base = ebase + cc * EB

                @pl.when(cc >= 2)
                def _(j1=j1):
                    wait_scatter(sets[j1])

                @pl.when(cc < LAST)
                def _(j1=j1, base=base):
                    wait_idx(base + EB, sets[j1])
                wait_gather(sets[j])

                @pl.when(cc < LAST)
                def _(j1=j1):
                    fire_gather(sets[j1])
                shadow_scale(sets[j])

                @pl.when(cc < LAST - 2)
                def _(j=j, base=base):
                    fire_idx(base + 3 * EB, sets[j])
                fire_scatter(sets[j])
            return carry

        lax.fori_loop(0, KK, step, 0)
        wait_scatter(sets[(LAST - 1) % 3])
        wait_scatter(sets[LAST % 3])
        plsc.subcore_barrier()

        for sz, cond in ((SLAB, s < NS - 1), (SLAB_LAST, s == NS - 1)):
            @pl.when(cond)
            def _(sz=sz):
                pltpu.sync_copy(acc.at[pl.ds(r0, sz)],
                                s_out.at[c, pl.ds(r0, sz)])

    return agg


def _make_cnt():
    mesh = plsc.VectorSubcoreMesh(
        core_axis_name="c", subcore_axis_name="s", num_cores=NC, num_subcores=NS)
    out_type = jax.ShapeDtypeStruct((NC, N, CW), jnp.float32)
    scratch = [
        pltpu.VMEM((CNT_B,), jnp.int32),
        pltpu.VMEM((CNT_B, CW), jnp.float32),
        pltpu.VMEM_SHARED((N, CW), jnp.float32),
    ]

    @functools.partial(pl.kernel, out_type=out_type, mesh=mesh,
                       scratch_types=scratch,
                       compiler_params=pltpu.CompilerParams(
                           use_tc_tiling_on_sc=False))
    def cnt_k(dst_e, z16, ones_in, cnt_out, dst_v, ones_v, cnt_acc):
        c = lax.axis_index("c")
        s = lax.axis_index("s")
        r0 = s * SLAB

        for sz, cond in ((SLAB, s < NS - 1), (SLAB_LAST, s == NS - 1)):
            @pl.when(cond)
            def _(sz=sz):
                pltpu.sync_copy(z16.at[pl.ds(r0, sz)],
                                cnt_acc.at[pl.ds(r0, sz)])
        pltpu.sync_copy(ones_in, ones_v)
        plsc.subcore_barrier()

        ebase = (s * NC + c) * CNT_E_PER_TILE

        def chunk(k, carry):
            pltpu.sync_copy(dst_e.at[pl.ds(ebase + k * CNT_B, CNT_B)], dst_v)
            pltpu.sync_copy(ones_v, cnt_acc.at[dst_v], add=True)
            return carry

        lax.fori_loop(0, CNT_CHUNKS, chunk, 0)
        plsc.subcore_barrier()

        for sz, cond in ((SLAB, s < NS - 1), (SLAB_LAST, s == NS - 1)):
            @pl.when(cond)
            def _(sz=sz):
                pltpu.sync_copy(cnt_acc.at[pl.ds(r0, sz)],
                                cnt_out.at[c, pl.ds(r0, sz)])

    return cnt_k


_agg_call = functools.lru_cache(maxsize=None)(_make_agg)
_cnt_call = functools.lru_cache(maxsize=None)(_make_cnt)


def _dense1(s2, cnt16, x, WrT, WtT, b):
    def body(s_ref, cnt_ref, x_ref, wr_ref, wt_ref, b_ref, h_ref):
        u = jnp.dot(s_ref[0], wr_ref[:DH], preferred_element_type=jnp.float32)
        u += jnp.dot(s_ref[1], wr_ref[DH:], preferred_element_type=jnp.float32)
        inv = 1.0 / jnp.maximum(cnt_ref[0, :, 0] + cnt_ref[1, :, 0], 1.0)
        u = u * inv[:, None] + b_ref[0][None, :]
        u += jnp.dot(x_ref[...], wt_ref[...], preferred_element_type=jnp.float32)
        h = jnp.maximum(u, 0.0)
        h_ref[0] = h[:, :DH]
        h_ref[1] = h[:, DH:]

    return pl.pallas_call(
        body,
        grid=(N // BN,),
        in_specs=[
            pl.BlockSpec((2, BN, DH), lambda i: (0, i, 0)),
            pl.BlockSpec((2, BN, CW), lambda i: (0, i, 0)),
            pl.BlockSpec((BN, D), lambda i: (i, 0)),
            pl.BlockSpec((D, D), lambda i: (0, 0)),
            pl.BlockSpec((D, D), lambda i: (0, 0)),
            pl.BlockSpec((1, D), lambda i: (0, 0)),
        ],
        out_specs=pl.BlockSpec((2, BN, DH), lambda i: (0, i, 0)),
        out_shape=jax.ShapeDtypeStruct((2, N, DH), jnp.float32),
    )(s2, cnt16, x, WrT, WtT, b)


def _dense2(s2, cnt16, h2, WrT, WtT, b, WlT, bl):
    def body(s_ref, cnt_ref, h_ref, wr_ref, wt_ref, b_ref, wl_ref, bl_ref,
             y_ref, o_ref):
        u = jnp.dot(s_ref[0], wr_ref[:DH], preferred_element_type=jnp.float32)
        u += jnp.dot(s_ref[1], wr_ref[DH:], preferred_element_type=jnp.float32)
        inv = 1.0 / jnp.maximum(cnt_ref[0, :, 0] + cnt_ref[1, :, 0], 1.0)
        u = u * inv[:, None] + b_ref[0][None, :]
        u += jnp.dot(h_ref[0], wt_ref[:DH], preferred_element_type=jnp.float32)
        u += jnp.dot(h_ref[1], wt_ref[DH:], preferred_element_type=jnp.float32)
        y = jnp.maximum(u, 0.0)
        y_ref[...] = y
        o = jnp.dot(y, wl_ref[...], preferred_element_type=jnp.float32)
        o_ref[...] = jax.nn.sigmoid(o + bl_ref[0][None, :])

    return pl.pallas_call(
        body,
        grid=(N // BN,),
        in_specs=[
            pl.BlockSpec((2, BN, DH), lambda i: (0, i, 0)),
            pl.BlockSpec((2, BN, CW), lambda i: (0, i, 0)),
            pl.BlockSpec((2, BN, DH), lambda i: (0, i, 0)),
            pl.BlockSpec((D, D), lambda i: (0, 0)),
            pl.BlockSpec((D, D), lambda i: (0, 0)),
            pl.BlockSpec((1, D), lambda i: (0, 0)),
            pl.BlockSpec((D, OUT), lambda i: (0, 0)),
            pl.BlockSpec((1, OUT), lambda i: (0, 0)),
        ],
        out_specs=[
            pl.BlockSpec((BN, D), lambda i: (i, 0)),
            pl.BlockSpec((BN, OUT), lambda i: (i, 0)),
        ],
        out_shape=[
            jax.ShapeDtypeStruct((N, D), jnp.float32),
            jax.ShapeDtypeStruct((N, OUT), jnp.float32),
        ],
    )(s2, cnt16, h2, WrT, WtT, b, WlT, bl)


def kernel(x, edge_index, edge_weight, W_rel1, b_rel1, W_root1,
           W_rel2, b_rel2, W_root2, W_lin, b_lin):
    xs0 = x[:, :DH]
    xs1 = x[:, DH:]
    src_e = edge_index[0]
    dst_e = edge_index[1]
    z2d = jnp.zeros((N, DH), jnp.float32)
    z16 = jnp.zeros((N, CW), jnp.float32)
    ones_in = jnp.ones((CNT_B, CW), jnp.float32)

    cnt16 = _cnt_call()(dst_e, z16, ones_in)
    pad = E_PAD - E
    srcp = jnp.concatenate([src_e, jnp.zeros((pad,), jnp.int32)])
    dstp = jnp.concatenate([dst_e, jnp.zeros((pad,), jnp.int32)])
    ewrp = jnp.concatenate(
        [jnp.broadcast_to(edge_weight[:, None], (E, 16)),
         jnp.zeros((pad, 16), jnp.float32)])
    s1 = _agg_call()(xs0, xs1, srcp, dstp, ewrp, z2d)
    h2 = _dense1(s1, cnt16, x, W_rel1.T, W_root1.T, b_rel1[None, :])
    s2 = _agg_call()(h2[0], h2[1], srcp, dstp, ewrp, z2d)
    y, out = _dense2(s2, cnt16, h2, W_rel2.T, W_root2.T, b_rel2[None, :],
                     W_lin.T, b_lin[None, :])
    return (out, y)


# speedup vs baseline: 1.1362x; 1.1362x over previous
import functools

import jax
import jax.numpy as jnp
from jax import lax
from jax.experimental import pallas as pl
from jax.experimental.pallas import tpu as pltpu
from jax.experimental.pallas import tpu_sc as plsc

N = 10000
E = 160000
D = 384
DH = D // 2
OUT = 128
NC = 2
NS = 16
EB = 24
EPT = 10032
CHUNKS = EPT // EB
KK = CHUNKS // 2
E_PAD = NS * EPT
SLAB = 632
SLAB_LAST = N - (NS - 1) * SLAB
CNT_B = 40
CW = 8
CNT_E_PER_TILE = E // (NC * NS)
CNT_CHUNKS = CNT_E_PER_TILE // CNT_B
BN = 1000


def _make_agg():
    mesh = plsc.VectorSubcoreMesh(
        core_axis_name="c", subcore_axis_name="s", num_cores=NC, num_subcores=NS)
    out_type = jax.ShapeDtypeStruct((NC, N, DH), jnp.float32)
    bufset = [
        pltpu.VMEM((EB,), jnp.int32),
        pltpu.VMEM((EB,), jnp.int32),
        pltpu.VMEM((EB, 16), jnp.float32),
        pltpu.VMEM((EB, DH), jnp.float32),
        pltpu.VMEM((EB,), jnp.int32),
        pltpu.SemaphoreType.DMA,
        pltpu.SemaphoreType.DMA,
        pltpu.SemaphoreType.DMA,
    ]
    scratch = bufset + bufset + [pltpu.VMEM_SHARED((N, DH), jnp.float32)]

    @functools.partial(pl.kernel, out_type=out_type, mesh=mesh,
                       scratch_types=scratch,
                       compiler_params=pltpu.CompilerParams(
                           use_tc_tiling_on_sc=False,
                           needs_layout_passes=False))
    def agg(xs0, xs1, src_e, dst_e, ewr, z2d, s_out,
            srcA, dstA, ewA, rowsA, shdA, semIA, semGA, semSA,
            srcB, dstB, ewB, rowsB, shdB, semIB, semGB, semSB, acc):
        c = lax.axis_index("c")
        s = lax.axis_index("s")
        r0 = s * SLAB

        for sz, cond in ((SLAB, s < NS - 1), (SLAB_LAST, s == NS - 1)):
            @pl.when(cond)
            def _(sz=sz):
                pltpu.sync_copy(z2d.at[pl.ds(r0, sz)], acc.at[pl.ds(r0, sz)])
        plsc.subcore_barrier()

        ebase = s * EPT

        def fire_idx(base, src_v, dst_v, ew_v, semI):
            pltpu.async_copy(src_e.at[pl.ds(base, EB)], src_v, semI)
            pltpu.async_copy(dst_e.at[pl.ds(base, EB)], dst_v, semI)
            pltpu.async_copy(ewr.at[pl.ds(base, EB)], ew_v, semI)

        def wait_idx(base, src_v, dst_v, ew_v, semI):
            pltpu.make_async_copy(src_e.at[pl.ds(base, EB)], src_v, semI).wait()
            pltpu.make_async_copy(dst_e.at[pl.ds(base, EB)], dst_v, semI).wait()
            pltpu.make_async_copy(ewr.at[pl.ds(base, EB)], ew_v, semI).wait()

        def fire_gather(src_v, rows_v, semG):
            @pl.when(c == 0)
            def _():
                pltpu.async_copy(xs0.at[src_v], rows_v, semG)

            @pl.when(c == 1)
            def _():
                pltpu.async_copy(xs1.at[src_v], rows_v, semG)

        def shadow_scale(dst_v, shd_v, ew_v, rows_v):
            shd_v[pl.ds(0, 16)] = dst_v[pl.ds(0, 16)]
            shd_v[pl.ds(8, 16)] = dst_v[pl.ds(8, 16)]
            for r in range(EB):
                m = ew_v[r, :]
                for q in range(DH // 16):
                    cs = pl.ds(q * 16, 16)
                    rows_v[r, cs] = rows_v[r, cs] * m

        fire_idx(ebase, srcA, dstA, ewA, semIA)
        fire_idx(ebase + EB, srcB, dstB, ewB, semIB)
        wait_idx(ebase, srcA, dstA, ewA, semIA)
        fire_gather(srcA, rowsA, semGA)

        def step(kk, carry):
            base = ebase + kk * (2 * EB)
            @pl.when(kk >= 1)
            def _():
                pltpu.make_async_copy(rowsB, acc.at[shdB], semSB).wait()
            wait_idx(base + EB, srcB, dstB, ewB, semIB)
            pltpu.make_async_copy(xs0.at[srcA], rowsA, semGA).wait()
            fire_gather(srcB, rowsB, semGB)
            shadow_scale(dstA, shdA, ewA, rowsA)
            @pl.when(kk < KK - 1)
            def _():
                fire_idx(base + 2 * EB, srcA, dstA, ewA, semIA)
            pltpu.async_copy(rowsA, acc.at[shdA], semSA, add=True)

            pltpu.make_async_copy(xs0.at[srcB], rowsB, semGB).wait()
            shadow_scale(dstB, shdB, ewB, rowsB)
            @pl.when(kk < KK - 1)
            def _():
                fire_idx(base + 3 * EB, srcB, dstB, ewB, semIB)
            pltpu.async_copy(rowsB, acc.at[shdB], semSB, add=True)

            @pl.when(kk < KK - 1)
            def _():
                pltpu.make_async_copy(rowsA, acc.at[shdA], semSA).wait()
                wait_idx(base + 2 * EB, srcA, dstA, ewA, semIA)
                fire_gather(srcA, rowsA, semGA)
            return carry

        lax.fori_loop(0, KK, step, 0)
        pltpu.make_async_copy(rowsA, acc.at[shdA], semSA).wait()
        pltpu.make_async_copy(rowsB, acc.at[shdB], semSB).wait()
        plsc.subcore_barrier()

        for sz, cond in ((SLAB, s < NS - 1), (SLAB_LAST, s == NS - 1)):
            @pl.when(cond)
            def _(sz=sz):
                pltpu.sync_copy(acc.at[pl.ds(r0, sz)],
                                s_out.at[c, pl.ds(r0, sz)])

    return agg


def _make_cnt():
    mesh = plsc.VectorSubcoreMesh(
        core_axis_name="c", subcore_axis_name="s", num_cores=NC, num_subcores=NS)
    out_type = jax.ShapeDtypeStruct((NC, N, CW), jnp.float32)
    scratch = [
        pltpu.VMEM((CNT_B,), jnp.int32),
        pltpu.VMEM((CNT_B, CW), jnp.float32),
        pltpu.VMEM_SHARED((N, CW), jnp.float32),
    ]

    @functools.partial(pl.kernel, out_type=out_type, mesh=mesh,
                       scratch_types=scratch,
                       compiler_params=pltpu.CompilerParams(
                           use_tc_tiling_on_sc=False))
    def cnt_k(dst_e, z16, ones_in, cnt_out, dst_v, ones_v, cnt_acc):
        c = lax.axis_index("c")
        s = lax.axis_index("s")
        r0 = s * SLAB

        for sz, cond in ((SLAB, s < NS - 1), (SLAB_LAST, s == NS - 1)):
            @pl.when(cond)
            def _(sz=sz):
                pltpu.sync_copy(z16.at[pl.ds(r0, sz)],
                                cnt_acc.at[pl.ds(r0, sz)])
        pltpu.sync_copy(ones_in, ones_v)
        plsc.subcore_barrier()

        ebase = (s * NC + c) * CNT_E_PER_TILE

        def chunk(k, carry):
            pltpu.sync_copy(dst_e.at[pl.ds(ebase + k * CNT_B, CNT_B)], dst_v)
            pltpu.sync_copy(ones_v, cnt_acc.at[dst_v], add=True)
            return carry

        lax.fori_loop(0, CNT_CHUNKS, chunk, 0)
        plsc.subcore_barrier()

        for sz, cond in ((SLAB, s < NS - 1), (SLAB_LAST, s == NS - 1)):
            @pl.when(cond)
            def _(sz=sz):
                pltpu.sync_copy(cnt_acc.at[pl.ds(r0, sz)],
                                cnt_out.at[c, pl.ds(r0, sz)])

    return cnt_k


_agg_call = functools.lru_cache(maxsize=None)(_make_agg)
_cnt_call = functools.lru_cache(maxsize=None)(_make_cnt)


def _dense1(s2, cnt16, x, WrT, WtT, b):
    def body(s_ref, cnt_ref, x_ref, wr_ref, wt_ref, b_ref, h_ref):
        u = jnp.dot(s_ref[0], wr_ref[:DH], preferred_element_type=jnp.float32)
        u += jnp.dot(s_ref[1], wr_ref[DH:], preferred_element_type=jnp.float32)
        inv = 1.0 / jnp.maximum(cnt_ref[0, :, 0] + cnt_ref[1, :, 0], 1.0)
        u = u * inv[:, None] + b_ref[0][None, :]
        u += jnp.dot(x_ref[...], wt_ref[...], preferred_element_type=jnp.float32)
        h = jnp.maximum(u, 0.0)
        h_ref[0] = h[:, :DH]
        h_ref[1] = h[:, DH:]

    return pl.pallas_call(
        body,
        grid=(N // BN,),
        in_specs=[
            pl.BlockSpec((2, BN, DH), lambda i: (0, i, 0)),
            pl.BlockSpec((2, BN, CW), lambda i: (0, i, 0)),
            pl.BlockSpec((BN, D), lambda i: (i, 0)),
            pl.BlockSpec((D, D), lambda i: (0, 0)),
            pl.BlockSpec((D, D), lambda i: (0, 0)),
            pl.BlockSpec((1, D), lambda i: (0, 0)),
        ],
        out_specs=pl.BlockSpec((2, BN, DH), lambda i: (0, i, 0)),
        out_shape=jax.ShapeDtypeStruct((2, N, DH), jnp.float32),
    )(s2, cnt16, x, WrT, WtT, b)


def _dense2(s2, cnt16, h2, WrT, WtT, b, WlT, bl):
    def body(s_ref, cnt_ref, h_ref, wr_ref, wt_ref, b_ref, wl_ref, bl_ref,
             y_ref, o_ref):
        u = jnp.dot(s_ref[0], wr_ref[:DH], preferred_element_type=jnp.float32)
        u += jnp.dot(s_ref[1], wr_ref[DH:], preferred_element_type=jnp.float32)
        inv = 1.0 / jnp.maximum(cnt_ref[0, :, 0] + cnt_ref[1, :, 0], 1.0)
        u = u * inv[:, None] + b_ref[0][None, :]
        u += jnp.dot(h_ref[0], wt_ref[:DH], preferred_element_type=jnp.float32)
        u += jnp.dot(h_ref[1], wt_ref[DH:], preferred_element_type=jnp.float32)
        y = jnp.maximum(u, 0.0)
        y_ref[...] = y
        o = jnp.dot(y, wl_ref[...], preferred_element_type=jnp.float32)
        o_ref[...] = jax.nn.sigmoid(o + bl_ref[0][None, :])

    return pl.pallas_call(
        body,
        grid=(N // BN,),
        in_specs=[
            pl.BlockSpec((2, BN, DH), lambda i: (0, i, 0)),
            pl.BlockSpec((2, BN, CW), lambda i: (0, i, 0)),
            pl.BlockSpec((2, BN, DH), lambda i: (0, i, 0)),
            pl.BlockSpec((D, D), lambda i: (0, 0)),
            pl.BlockSpec((D, D), lambda i: (0, 0)),
            pl.BlockSpec((1, D), lambda i: (0, 0)),
            pl.BlockSpec((D, OUT), lambda i: (0, 0)),
            pl.BlockSpec((1, OUT), lambda i: (0, 0)),
        ],
        out_specs=[
            pl.BlockSpec((BN, D), lambda i: (i, 0)),
            pl.BlockSpec((BN, OUT), lambda i: (i, 0)),
        ],
        out_shape=[
            jax.ShapeDtypeStruct((N, D), jnp.float32),
            jax.ShapeDtypeStruct((N, OUT), jnp.float32),
        ],
    )(s2, cnt16, h2, WrT, WtT, b, WlT, bl)


def kernel(x, edge_index, edge_weight, W_rel1, b_rel1, W_root1,
           W_rel2, b_rel2, W_root2, W_lin, b_lin):
    xs0 = x[:, :DH]
    xs1 = x[:, DH:]
    src_e = edge_index[0]
    dst_e = edge_index[1]
    z2d = jnp.zeros((N, DH), jnp.float32)
    z16 = jnp.zeros((N, CW), jnp.float32)
    ones_in = jnp.ones((CNT_B, CW), jnp.float32)

    cnt16 = _cnt_call()(dst_e, z16, ones_in)
    pad = E_PAD - E
    srcp = jnp.concatenate([src_e, jnp.zeros((pad,), jnp.int32)])
    dstp = jnp.concatenate([dst_e, jnp.zeros((pad,), jnp.int32)])
    ewrp = jnp.concatenate(
        [jnp.broadcast_to(edge_weight[:, None], (E, 16)),
         jnp.zeros((pad, 16), jnp.float32)])
    s1 = _agg_call()(xs0, xs1, srcp, dstp, ewrp, z2d)
    h2 = _dense1(s1, cnt16, x, W_rel1.T, W_root1.T, b_rel1[None, :])
    s2 = _agg_call()(h2[0], h2[1], srcp, dstp, ewrp, z2d)
    y, out = _dense2(s2, cnt16, h2, W_rel2.T, W_root2.T, b_rel2[None, :],
                     W_lin.T, b_lin[None, :])
    return (out, y)
